# baseline (device time: 8627 ns/iter reference)
import jax
import jax.numpy as jnp
from jax import lax
from jax.experimental import pallas as pl
from jax.experimental.pallas import tpu as pltpu

N_DEV = 4
EPS = 1e-5


def kernel(x, gamma):
    m, n_per = x.shape
    n_global = N_DEV * n_per

    def body(x_hbm, g_ref, out_ref, xv_ref, gx_ref, comm_ref,
             in_sem, send_sems, recv_sems):
        my = lax.axis_index("i")

        barrier_sem = pltpu.get_barrier_semaphore()
        for d in range(1, N_DEV):
            pl.semaphore_signal(
                barrier_sem, inc=1,
                device_id=((my + d) % N_DEV,),
                device_id_type=pl.DeviceIdType.MESH,
            )

        cp = pltpu.make_async_copy(x_hbm, xv_ref, in_sem)
        cp.start()
        cp.wait()

        xv = xv_ref[:, :]
        p = jnp.sum(xv * xv, axis=1, keepdims=True)
        comm_ref[my] = jnp.transpose(p, (1, 0))

        pl.semaphore_wait(barrier_sem, N_DEV - 1)

        sends = []
        for d in (2, 1, 3):
            rdma = pltpu.make_async_remote_copy(
                src_ref=comm_ref.at[my],
                dst_ref=comm_ref.at[my],
                send_sem=send_sems.at[d - 1],
                recv_sem=recv_sems.at[my],
                device_id=((my + d) % N_DEV,),
                device_id_type=pl.DeviceIdType.MESH,
            )
            rdma.start()
            sends.append(rdma)

        gx_ref[:, :] = jnp.reshape(g_ref[:], (1, n_per)) * xv

        for d in range(1, N_DEV):
            sender = (my + d) % N_DEV
            recv = pltpu.make_async_remote_copy(
                src_ref=comm_ref.at[my],
                dst_ref=comm_ref.at[sender],
                send_sem=send_sems.at[d - 1],
                recv_sem=recv_sems.at[sender],
                device_id=(sender,),
                device_id_type=pl.DeviceIdType.MESH,
            )
            recv.wait_recv()
        for rdma in sends:
            rdma.wait_send()

        total = comm_ref[0]
        for s in range(1, N_DEV):
            total = total + comm_ref[s]
        inv = jnp.transpose(lax.rsqrt(total / n_global + EPS), (1, 0))
        out_ref[:, :] = gx_ref[:, :] * inv

    return pl.pallas_call(
        body,
        out_shape=jax.ShapeDtypeStruct((m, n_per), jnp.float32),
        in_specs=[
            pl.BlockSpec(memory_space=pl.ANY),
            pl.BlockSpec(memory_space=pltpu.VMEM),
        ],
        out_specs=pl.BlockSpec(memory_space=pltpu.VMEM),
        scratch_shapes=[
            pltpu.VMEM((m, n_per), jnp.float32),
            pltpu.VMEM((m, n_per), jnp.float32),
            pltpu.VMEM((N_DEV, 1, m), jnp.float32),
            pltpu.SemaphoreType.DMA,
            pltpu.SemaphoreType.DMA((N_DEV - 1,)),
            pltpu.SemaphoreType.DMA((N_DEV,)),
        ],
        compiler_params=pltpu.CompilerParams(collective_id=0),
    )(x, gamma)
